# Initial kernel scaffold; baseline (speedup 1.0000x reference)
import jax
import jax.numpy as jnp
from jax.experimental import pallas as pl


def kernel(x, params, edge_index, batch):
    raise NotImplementedError("write your pallas kernel here")


# speedup vs baseline: 32.4500x; 32.4500x over previous
import functools

import jax
import jax.numpy as jnp
from jax import lax
from jax.experimental import pallas as pl
from jax.experimental.pallas import tpu as pltpu
from jax.experimental.pallas import tpu_sc as plsc

N = 50000
E = 800000
BSZ = 16
H = 64
HEADS = 8
HEAD_DIM = 8

NC = 2
NS = 16
WB = 3128
WB_LAST = N - (NS - 1) * WB
ECHUNK = 125
EROWS = E // ECHUNK

_MESH = plsc.VectorSubcoreMesh(
    core_axis_name="c", subcore_axis_name="s", num_cores=NC, num_subcores=NS)


def _zero_shared(zbuf, shared, s):
  base = s * WB

  def zcopy(i, _):
    pltpu.sync_copy(zbuf, shared.at[pl.ds(base + i * 128, 128)])
    return 0
  lax.fori_loop(0, 24, zcopy, 0)

  @pl.when(s < NS - 1)
  def _():
    pltpu.sync_copy(zbuf.at[pl.ds(0, WB - 3072)],
                    shared.at[pl.ds(base + 3072, WB - 3072)])

  @pl.when(s == NS - 1)
  def _():
    pltpu.sync_copy(zbuf.at[pl.ds(0, WB_LAST - 3072)],
                    shared.at[pl.ds(base + 3072, WB_LAST - 3072)])


def _writeback(shared, out, c, s):
  base = s * WB

  @pl.when(s < NS - 1)
  def _():
    pltpu.sync_copy(shared.at[pl.ds(base, WB)],
                    out.at[pl.ds(c * N + base, WB)])

  @pl.when(s == NS - 1)
  def _():
    pltpu.sync_copy(shared.at[pl.ds(base, WB_LAST)],
                    out.at[pl.ds(c * N + base, WB_LAST)])


def _deg_body(dst2d, cnt2, shared, dstbuf, ones, zbuf):
  c = lax.axis_index("c")
  s = lax.axis_index("s")

  def fill(r, _):
    ones[r] = jnp.full((16,), 1.0, jnp.float32)
    zbuf[r] = jnp.zeros((16,), jnp.float32)
    return 0
  lax.fori_loop(0, 128, fill, 0)

  _zero_shared(zbuf, shared, s)
  plsc.subcore_barrier()

  w = c * NS + s
  rows_per_worker = EROWS // (NC * NS)

  def group(g8, _):
    base = w * rows_per_worker + g8 * 8
    pltpu.sync_copy(dst2d.at[pl.ds(base, 8)], dstbuf)
    for j in range(8):
      pltpu.sync_copy(ones.at[pl.ds(0, ECHUNK)],
                      shared.at[dstbuf.at[j]], add=True)
    return 0
  lax.fori_loop(0, rows_per_worker // 8, group, 0)
  plsc.subcore_barrier()
  _writeback(shared, cnt2, c, s)


_deg_call = functools.partial(
    pl.kernel, _deg_body,
    out_type=jax.ShapeDtypeStruct((NC * N, 16), jnp.float32),
    mesh=_MESH,
    compiler_params=pltpu.CompilerParams(use_tc_tiling_on_sc=False),
    scratch_types=[
        pltpu.VMEM_SHARED((N, 16), jnp.float32),
        pltpu.VMEM((8, ECHUNK), jnp.int32),
        pltpu.VMEM((128, 16), jnp.float32),
        pltpu.VMEM((128, 16), jnp.float32),
    ])


def _segsum_body(yA, yB, src2d, dst2d, S2, shared, srcbuf, dstbuf, rows,
                 zbuf, sem):
  c = lax.axis_index("c")
  s = lax.axis_index("s")

  def fill(r, _):
    for k in range(2):
      zbuf[r, pl.ds(16 * k, 16)] = jnp.zeros((16,), jnp.float32)
    return 0
  lax.fori_loop(0, 128, fill, 0)

  _zero_shared(zbuf, shared, s)
  plsc.subcore_barrier()

  rows_per_tile_edges = EROWS // NS

  def edge_loop(y):
    def group(g8, _):
      base = s * rows_per_tile_edges + g8 * 8
      pltpu.sync_copy(src2d.at[pl.ds(base, 8)], srcbuf)
      pltpu.sync_copy(dst2d.at[pl.ds(base, 8)], dstbuf)
      for j in range(8):
        pltpu.async_copy(y.at[srcbuf.at[j]], rows, sem).wait()
        pltpu.sync_copy(rows, shared.at[dstbuf.at[j]], add=True)
      return 0
    lax.fori_loop(0, rows_per_tile_edges // 8, group, 0)

  @pl.when(c == 0)
  def _():
    edge_loop(yA)

  @pl.when(c == 1)
  def _():
    edge_loop(yB)

  plsc.subcore_barrier()
  _writeback(shared, S2, c, s)


_segsum_call = functools.partial(
    pl.kernel, _segsum_body,
    out_type=jax.ShapeDtypeStruct((NC * N, 32), jnp.float32),
    mesh=_MESH,
    compiler_params=pltpu.CompilerParams(use_tc_tiling_on_sc=False),
    scratch_types=[
        pltpu.VMEM_SHARED((N, 32), jnp.float32),
        pltpu.VMEM((8, ECHUNK), jnp.int32),
        pltpu.VMEM((8, ECHUNK), jnp.int32),
        pltpu.VMEM((ECHUNK, 32), jnp.float32),
        pltpu.VMEM((128, 32), jnp.float32),
        pltpu.SemaphoreType.DMA,
    ])


def _gden_body(a16s, a16d, src2d, dst2d, den2, shden, srcbuf, dstbuf,
               asb, adb, ebuf, zbuf, sem):
  c = lax.axis_index("c")
  s = lax.axis_index("s")

  def fill(r, _):
    zbuf[r] = jnp.zeros((16,), jnp.float32)
    return 0
  lax.fori_loop(0, 128, fill, 0)

  _zero_shared(zbuf, shden, s)
  plsc.subcore_barrier()

  w = c * NS + s
  rows_per_worker = EROWS // (NC * NS)

  def group(g8, _):
    base = w * rows_per_worker + g8 * 8
    pltpu.sync_copy(src2d.at[pl.ds(base, 8)], srcbuf)
    pltpu.sync_copy(dst2d.at[pl.ds(base, 8)], dstbuf)
    for j in range(8):
      pltpu.async_copy(a16s.at[srcbuf.at[j]], asb, sem).wait()
      pltpu.async_copy(a16d.at[dstbuf.at[j]], adb, sem).wait()

      def ecomp(r, _):
        v = asb[r] + adb[r]
        ebuf[r] = jnp.exp(jnp.where(v > 0, v, 0.2 * v))
        return 0
      lax.fori_loop(0, ECHUNK, ecomp, 0)
      pltpu.sync_copy(ebuf, shden.at[dstbuf.at[j]], add=True)
    return 0
  lax.fori_loop(0, rows_per_worker // 8, group, 0)
  plsc.subcore_barrier()
  _writeback(shden, den2, c, s)


_gden_call = functools.partial(
    pl.kernel, _gden_body,
    out_type=jax.ShapeDtypeStruct((NC * N, 16), jnp.float32),
    mesh=_MESH,
    compiler_params=pltpu.CompilerParams(use_tc_tiling_on_sc=False),
    scratch_types=[
        pltpu.VMEM_SHARED((N, 16), jnp.float32),
        pltpu.VMEM((8, ECHUNK), jnp.int32),
        pltpu.VMEM((8, ECHUNK), jnp.int32),
        pltpu.VMEM((ECHUNK, 16), jnp.float32),
        pltpu.VMEM((ECHUNK, 16), jnp.float32),
        pltpu.VMEM((ECHUNK, 16), jnp.float32),
        pltpu.VMEM((128, 16), jnp.float32),
        pltpu.SemaphoreType.DMA,
    ])


def _gfeat_body(ase0, ade0, ase1, ade1, xlA, xlB, src2d, dst2d, raw2,
                shraw, srcbuf, dstbuf, asb, adb, xlb, msg, zbuf, sem):
  c = lax.axis_index("c")
  s = lax.axis_index("s")

  def fill(r, _):
    for k in range(2):
      zbuf[r, pl.ds(16 * k, 16)] = jnp.zeros((16,), jnp.float32)
    return 0
  lax.fori_loop(0, 128, fill, 0)

  _zero_shared(zbuf, shraw, s)
  plsc.subcore_barrier()

  rows_per_tile_edges = EROWS // NS

  def edge_loop(ase, ade, xl):
    def group(g8, _):
      base = s * rows_per_tile_edges + g8 * 8
      pltpu.sync_copy(src2d.at[pl.ds(base, 8)], srcbuf)
      pltpu.sync_copy(dst2d.at[pl.ds(base, 8)], dstbuf)
      for j in range(8):
        pltpu.async_copy(ase.at[srcbuf.at[j]], asb, sem).wait()
        pltpu.async_copy(ade.at[dstbuf.at[j]], adb, sem).wait()
        pltpu.async_copy(xl.at[srcbuf.at[j]], xlb, sem).wait()

        def feat(r, _):
          for k in range(2):
            sl = pl.ds(16 * k, 16)
            v = asb[r, sl] + adb[r, sl]
            ev = jnp.exp(jnp.where(v > 0, v, 0.2 * v))
            msg[r, sl] = xlb[r, sl] * ev
          return 0
        lax.fori_loop(0, ECHUNK, feat, 0)
        pltpu.sync_copy(msg, shraw.at[dstbuf.at[j]], add=True)
      return 0
    lax.fori_loop(0, rows_per_tile_edges // 8, group, 0)

  @pl.when(c == 0)
  def _():
    edge_loop(ase0, ade0, xlA)

  @pl.when(c == 1)
  def _():
    edge_loop(ase1, ade1, xlB)

  plsc.subcore_barrier()
  _writeback(shraw, raw2, c, s)


_gfeat_call = functools.partial(
    pl.kernel, _gfeat_body,
    out_type=jax.ShapeDtypeStruct((NC * N, 32), jnp.float32),
    mesh=_MESH,
    compiler_params=pltpu.CompilerParams(use_tc_tiling_on_sc=False),
    scratch_types=[
        pltpu.VMEM_SHARED((N, 32), jnp.float32),
        pltpu.VMEM((8, ECHUNK), jnp.int32),
        pltpu.VMEM((8, ECHUNK), jnp.int32),
        pltpu.VMEM((ECHUNK, 32), jnp.float32),
        pltpu.VMEM((ECHUNK, 32), jnp.float32),
        pltpu.VMEM((ECHUNK, 32), jnp.float32),
        pltpu.VMEM((ECHUNK, 32), jnp.float32),
        pltpu.VMEM((128, 32), jnp.float32),
        pltpu.SemaphoreType.DMA,
    ])


def kernel(x, params, edge_index, batch):
  n = N
  src2d = edge_index[0].reshape(EROWS, ECHUNK)
  dst2d = edge_index[1].reshape(EROWS, ECHUNK)

  cnt2 = _deg_call()(dst2d)
  cnt = cnt2[:N, 0] + cnt2[N:, 0]
  dinv = lax.rsqrt(jnp.maximum(cnt + 1.0, 1e-12))

  def segsum(y):
    S2 = _segsum_call()(y[:, :32], y[:, 32:], src2d, dst2d)
    return jnp.concatenate([S2[:N], S2[N:]], axis=1)

  def gcn(xin, W, b):
    y = dinv[:, None] * (xin @ W)
    return dinv[:, None] * (segsum(y) + y) + b

  def gnorm(h, g, bb, a):
    mean = h.mean(0)
    var = (h * h).mean(0) - (2 * a - a * a) * mean * mean
    return g * (h - a * mean) / jnp.sqrt(var + 1e-5) + bb

  fan_in, fan_out, degc, ior, pr, bt, cl = [x[:, i] for i in range(7)]
  hub_scores = jnp.clip(
      0.25 * jax.nn.sigmoid(fan_in + fan_out - 3.0)
      + 0.2 * jax.nn.sigmoid(fan_out - 2.0)
      + 0.15 * jnp.clip(1.0 - jnp.abs(ior - 1.0), 0.0, 1.0)
      + 0.15 * pr / (pr.max() + 1e-8)
      + 0.1 * bt / (bt.max() + 1e-8)
      + 0.1 * cl / (cl.max() + 1e-8)
      + 0.05 * degc, 0.0, 1.0)

  mask = jnp.zeros((n,), bool)
  iota = jnp.arange(n)
  for b in range(BSZ):
    work = jnp.where(batch == b, hub_scores, -jnp.inf)
    for _ in range(3):
      v = work.max()
      i = jnp.where(work == v, iota, n).min()
      mask = mask.at[i].set(True)
      work = work.at[i].set(-jnp.inf)

  x_emb = x @ params['W_embed'] + params['b_embed']
  h = gcn(x_emb, params['W_g0'], params['b_g0'])
  h = gnorm(h, params['gn0_g'], params['gn0_b'], params['gn0_a'])
  x_emb = jax.nn.relu(h) + x_emb

  xl = x_emb @ params['W_a1']
  xlh = xl.reshape(n, HEADS, HEAD_DIM)
  a_s = (xlh * params['att_src']).sum(-1)
  a_d = (xlh * params['att_dst']).sum(-1)
  pad8 = jnp.zeros((n, 8), jnp.float32)
  a16s = jnp.concatenate([a_s, pad8], axis=1)
  a16d = jnp.concatenate([a_d, pad8], axis=1)
  ase = jnp.repeat(a_s, HEAD_DIM, axis=1)
  ade = jnp.repeat(a_d, HEAD_DIM, axis=1)
  den2 = _gden_call()(a16s, a16d, src2d, dst2d)
  raw2 = _gfeat_call()(ase[:, :32], ade[:, :32], ase[:, 32:], ade[:, 32:],
                       xl[:, :32], xl[:, 32:], src2d, dst2d)
  raw = jnp.concatenate([raw2[:N], raw2[N:]], axis=1).reshape(n, HEADS,
                                                              HEAD_DIM)
  den = den2[:N, :8] + den2[N:, :8]
  lrelu = lambda v: jnp.where(v > 0, v, 0.2 * v)
  e_self = jnp.exp(lrelu(a_s + a_d))
  den = den + e_self
  raw = raw + e_self[:, :, None] * xlh
  gat = (raw / (den[:, :, None] + 1e-16)).reshape(n, H) + params['b_a1']
  h = gnorm(gat, params['gn1_g'], params['gn1_b'], params['gn1_a'])
  x_emb = jax.nn.relu(h) + x_emb

  h = gcn(x_emb, params['W_g2'], params['b_g2'])
  h = gnorm(h, params['gn2_g'], params['gn2_b'], params['gn2_a'])
  x_emb = jax.nn.relu(h) + x_emb

  hub_input = jnp.concatenate([x_emb, x[:, :7]], axis=-1)
  h1 = jax.nn.relu(hub_input @ params['hi_W1'] + params['hi_b1'])
  h2 = jax.nn.relu(h1 @ params['hi_W2'] + params['hi_b2'])
  learned = jax.nn.sigmoid((h2 @ params['hi_W3'] + params['hi_b3']).squeeze(-1))
  combined = 0.6 * learned + 0.4 * hub_scores
  hub_feat = jnp.concatenate([x_emb, combined[:, None], hub_scores[:, None]],
                             axis=-1)
  s1 = jax.nn.relu(hub_feat @ params['sel_W1'] + params['sel_b1'])
  hub_logits = (s1 @ params['sel_W2'] + params['sel_b2']).squeeze(-1)
  noise = 0.15 * 0.1 * jax.random.normal(jax.random.key(1234),
                                         hub_logits.shape, hub_logits.dtype)
  hub_logits = (hub_logits + 3.0 * hub_scores + noise
                + 2.0 * mask.astype(x.dtype))

  m = jax.ops.segment_max(hub_logits, batch, num_segments=BSZ)
  eh = jnp.exp(hub_logits - m[batch])
  dh = jax.ops.segment_sum(eh, batch, num_segments=BSZ)
  hub_probs = eh / (dh[batch] + 1e-16)

  p1 = jax.nn.relu(hub_input @ params['pat_W1'] + params['pat_b1'])
  pattern_logits = (p1 @ params['pat_W2'] + params['pat_b2']
                    + hub_scores[:, None] * 0.5)
  pattern_probs = jax.nn.softmax(pattern_logits, axis=-1)

  cntb = jax.ops.segment_sum(jnp.ones((n,), jnp.float32), batch,
                             num_segments=BSZ)
  gmean = jax.ops.segment_sum(x_emb, batch, num_segments=BSZ) / cntb[:, None]
  gmax = jax.ops.segment_max(x_emb, batch, num_segments=BSZ)
  gfeat = jnp.concatenate([gmean, gmax], axis=-1)
  target_logits = jnp.zeros_like(hub_logits)
  t1 = jax.nn.relu(gfeat @ params['term_W1'] + params['term_b1'])
  term_logits = t1 @ params['term_W2'] + params['term_b2']
  term_probs = jax.nn.softmax(term_logits, axis=-1)
  v1 = jax.nn.relu(gfeat @ params['val_W1'] + params['val_b1'])
  value = v1 @ params['val_W2'] + params['val_b2']
  return (value, hub_logits, hub_probs, pattern_logits, pattern_probs,
          target_logits, term_logits, term_probs, combined, hub_scores,
          mask, x_emb)
